# Initial kernel scaffold; baseline (speedup 1.0000x reference)
import jax
import jax.numpy as jnp
from jax.experimental import pallas as pl


def kernel(text_features, spatial_features, body_features, rhythm_features, text_feature, spatial_feature, body_feature, rhythm_feature, motion_lengths, length):
    raise NotImplementedError("write your pallas kernel here")


# speedup vs baseline: 1.8162x; 1.8162x over previous
import functools

import jax
import jax.numpy as jnp
from jax.experimental import pallas as pl
from jax.experimental.pallas import tpu as pltpu

K = 100000
D = 768
BK = 800
NB = K // BK
NUM_RETRIEVAL = 4
KINEMATIC_COEF = 0.1
EPS = 1e-8

KP = 100352
TOPK_ROWS = KP // 128


def _score_kernel(lf_ref, t_ref, s_ref, b_ref, r_ref,
                  qt_ref, qs_ref, qb_ref, qr_ref, ml_ref, out_ref):
    def sim(x_ref, q_ref):
        x = x_ref[...]
        q = q_ref[...]
        dot = jnp.sum(x * q, axis=1, keepdims=True)
        nrm = jnp.sqrt(jnp.sum(x * x, axis=1, keepdims=True))
        qn = jnp.sqrt(jnp.sum(q * q))
        return dot / (jnp.maximum(nrm, EPS) * jnp.maximum(qn, EPS))

    semantic = (sim(t_ref, qt_ref) + sim(s_ref, qs_ref)
                + sim(b_ref, qb_ref) + sim(r_ref, qr_ref)) * 0.25

    lf = lf_ref[0, 0]
    ml = ml_ref[0].astype(jnp.float32)
    rel = jnp.abs(ml - lf) / jnp.maximum(ml, lf)
    kin = jnp.exp(-rel * KINEMATIC_COEF)
    out_ref[0] = semantic * kin


def _topk_kernel(x_ref, vals_ref, idxs_ref):
    x = x_ref[...]
    row = jax.lax.broadcasted_iota(jnp.int32, x.shape, 0)
    col = jax.lax.broadcasted_iota(jnp.int32, x.shape, 1)
    lin = row * 128 + col
    big = jnp.int32(2**31 - 1)
    for j in range(NUM_RETRIEVAL):
        m = jnp.max(x)
        idx = jnp.min(jnp.where(x == m, lin, big))
        vals_ref[j] = m
        idxs_ref[j] = idx
        x = jnp.where(lin == idx, -jnp.inf, x)


def kernel(text_features, spatial_features, body_features, rhythm_features,
           text_feature, spatial_feature, body_feature, rhythm_feature,
           motion_lengths, length):
    lf = jnp.asarray(length, jnp.float32).reshape(1, 1)
    ml3 = motion_lengths.reshape(NB, BK, 1)

    tbl_spec = pl.BlockSpec((BK, D), lambda b: (b, 0))
    q_spec = pl.BlockSpec((1, D), lambda b: (0, 0))

    scores = pl.pallas_call(
        _score_kernel,
        grid=(NB,),
        in_specs=[
            pl.BlockSpec(memory_space=pltpu.SMEM),
            tbl_spec, tbl_spec, tbl_spec, tbl_spec,
            q_spec, q_spec, q_spec, q_spec,
            pl.BlockSpec((1, BK, 1), lambda b: (b, 0, 0)),
        ],
        out_specs=pl.BlockSpec((1, BK, 1), lambda b: (b, 0, 0)),
        out_shape=jax.ShapeDtypeStruct((NB, BK, 1), jnp.float32),
    )(lf, text_features, spatial_features, body_features, rhythm_features,
      text_feature, spatial_feature, body_feature, rhythm_feature, ml3)

    combined_score = scores.reshape(K)

    padded = jnp.concatenate(
        [combined_score, jnp.full((KP - K,), -jnp.inf, jnp.float32)]
    ).reshape(TOPK_ROWS, 128)

    top_values, top_indices = pl.pallas_call(
        _topk_kernel,
        in_specs=[pl.BlockSpec((TOPK_ROWS, 128), lambda: (0, 0))],
        out_specs=[pl.BlockSpec(memory_space=pltpu.SMEM),
                   pl.BlockSpec(memory_space=pltpu.SMEM)],
        out_shape=[jax.ShapeDtypeStruct((NUM_RETRIEVAL,), jnp.float32),
                   jax.ShapeDtypeStruct((NUM_RETRIEVAL,), jnp.int32)],
    )(padded)

    return combined_score, top_values, top_indices


# speedup vs baseline: 1.8796x; 1.0349x over previous
import functools

import jax
import jax.numpy as jnp
from jax.experimental import pallas as pl
from jax.experimental.pallas import tpu as pltpu

K = 100000
D = 768
BK = 2000
NB = K // BK
NUM_RETRIEVAL = 4
KINEMATIC_COEF = 0.1
EPS = 1e-8

KP = 100352
TOPK_ROWS = KP // 128


def _score_kernel(lf_ref, t_ref, s_ref, b_ref, r_ref,
                  qt_ref, qs_ref, qb_ref, qr_ref, ml_ref, out_ref):
    def sim(x_ref, q_ref):
        x = x_ref[...]
        q = q_ref[...]
        dot = jnp.sum(x * q, axis=1, keepdims=True)
        nrm = jnp.sqrt(jnp.sum(x * x, axis=1, keepdims=True))
        qn = jnp.sqrt(jnp.sum(q * q))
        return dot / (jnp.maximum(nrm, EPS) * jnp.maximum(qn, EPS))

    semantic = (sim(t_ref, qt_ref) + sim(s_ref, qs_ref)
                + sim(b_ref, qb_ref) + sim(r_ref, qr_ref)) * 0.25

    lf = lf_ref[0, 0]
    ml = ml_ref[0].astype(jnp.float32)
    rel = jnp.abs(ml - lf) / jnp.maximum(ml, lf)
    kin = jnp.exp(-rel * KINEMATIC_COEF)
    out_ref[0] = semantic * kin


def _topk_kernel(x_ref, vals_ref, idxs_ref):
    x = x_ref[...]
    row = jax.lax.broadcasted_iota(jnp.int32, x.shape, 0)
    col = jax.lax.broadcasted_iota(jnp.int32, x.shape, 1)
    lin = row * 128 + col
    big = jnp.int32(2**31 - 1)
    for j in range(NUM_RETRIEVAL):
        m = jnp.max(x)
        idx = jnp.min(jnp.where(x == m, lin, big))
        vals_ref[j] = m
        idxs_ref[j] = idx
        x = jnp.where(lin == idx, -jnp.inf, x)


def kernel(text_features, spatial_features, body_features, rhythm_features,
           text_feature, spatial_feature, body_feature, rhythm_feature,
           motion_lengths, length):
    lf = jnp.asarray(length, jnp.float32).reshape(1, 1)
    ml3 = motion_lengths.reshape(NB, BK, 1)

    tbl_spec = pl.BlockSpec((BK, D), lambda b: (b, 0))
    q_spec = pl.BlockSpec((1, D), lambda b: (0, 0))

    scores = pl.pallas_call(
        _score_kernel,
        grid=(NB,),
        in_specs=[
            pl.BlockSpec(memory_space=pltpu.SMEM),
            tbl_spec, tbl_spec, tbl_spec, tbl_spec,
            q_spec, q_spec, q_spec, q_spec,
            pl.BlockSpec((1, BK, 1), lambda b: (b, 0, 0)),
        ],
        out_specs=pl.BlockSpec((1, BK, 1), lambda b: (b, 0, 0)),
        out_shape=jax.ShapeDtypeStruct((NB, BK, 1), jnp.float32),
    )(lf, text_features, spatial_features, body_features, rhythm_features,
      text_feature, spatial_feature, body_feature, rhythm_feature, ml3)

    combined_score = scores.reshape(K)

    padded = jnp.concatenate(
        [combined_score, jnp.full((KP - K,), -jnp.inf, jnp.float32)]
    ).reshape(TOPK_ROWS, 128)

    top_values, top_indices = pl.pallas_call(
        _topk_kernel,
        in_specs=[pl.BlockSpec((TOPK_ROWS, 128), lambda: (0, 0))],
        out_specs=[pl.BlockSpec(memory_space=pltpu.SMEM),
                   pl.BlockSpec(memory_space=pltpu.SMEM)],
        out_shape=[jax.ShapeDtypeStruct((NUM_RETRIEVAL,), jnp.float32),
                   jax.ShapeDtypeStruct((NUM_RETRIEVAL,), jnp.int32)],
    )(padded)

    return combined_score, top_values, top_indices


# speedup vs baseline: 2.3628x; 1.2571x over previous
import functools

import jax
import jax.numpy as jnp
from jax.experimental import pallas as pl
from jax.experimental.pallas import tpu as pltpu

K = 100000
D = 768
BK = 2000
NB = K // BK
NUM_RETRIEVAL = 4
KINEMATIC_COEF = 0.1
EPS = 1e-8

KP = 100352
TOPK_ROWS = KP // 128


def _score_kernel(lf_ref, t_ref, s_ref, b_ref, r_ref,
                  qt_ref, qs_ref, qb_ref, qr_ref, ml_ref, out_ref):
    def sim(x_ref, q_ref):
        x = x_ref[...]
        q = q_ref[...]
        dot = jnp.sum(x * q, axis=1, keepdims=True)
        nrm = jnp.sqrt(jnp.sum(x * x, axis=1, keepdims=True))
        qn = jnp.sqrt(jnp.sum(q * q))
        return dot / (jnp.maximum(nrm, EPS) * jnp.maximum(qn, EPS))

    semantic = (sim(t_ref, qt_ref) + sim(s_ref, qs_ref)
                + sim(b_ref, qb_ref) + sim(r_ref, qr_ref)) * 0.25

    lf = lf_ref[0, 0]
    ml = ml_ref[0].astype(jnp.float32)
    rel = jnp.abs(ml - lf) / jnp.maximum(ml, lf)
    kin = jnp.exp(-rel * KINEMATIC_COEF)
    out_ref[0] = jnp.transpose(semantic) * kin


def _topk_kernel(x_ref, vals_ref, idxs_ref):
    x = x_ref[...]
    row = jax.lax.broadcasted_iota(jnp.int32, x.shape, 0)
    col = jax.lax.broadcasted_iota(jnp.int32, x.shape, 1)
    lin = row * 128 + col
    big = jnp.int32(2**31 - 1)
    for j in range(NUM_RETRIEVAL):
        m = jnp.max(x)
        idx = jnp.min(jnp.where(x == m, lin, big))
        vals_ref[j] = m
        idxs_ref[j] = idx
        x = jnp.where(lin == idx, -jnp.inf, x)


def kernel(text_features, spatial_features, body_features, rhythm_features,
           text_feature, spatial_feature, body_feature, rhythm_feature,
           motion_lengths, length):
    lf = jnp.asarray(length, jnp.float32).reshape(1, 1)
    ml3 = motion_lengths.reshape(NB, 1, BK)

    tbl_spec = pl.BlockSpec((BK, D), lambda b: (b, 0))
    q_spec = pl.BlockSpec((1, D), lambda b: (0, 0))

    scores = pl.pallas_call(
        _score_kernel,
        grid=(NB,),
        in_specs=[
            pl.BlockSpec(memory_space=pltpu.SMEM),
            tbl_spec, tbl_spec, tbl_spec, tbl_spec,
            q_spec, q_spec, q_spec, q_spec,
            pl.BlockSpec((1, 1, BK), lambda b: (b, 0, 0)),
        ],
        out_specs=pl.BlockSpec((1, 1, BK), lambda b: (b, 0, 0)),
        out_shape=jax.ShapeDtypeStruct((NB, 1, BK), jnp.float32),
    )(lf, text_features, spatial_features, body_features, rhythm_features,
      text_feature, spatial_feature, body_feature, rhythm_feature, ml3)

    combined_score = scores.reshape(K)

    padded = jnp.concatenate(
        [combined_score, jnp.full((KP - K,), -jnp.inf, jnp.float32)]
    ).reshape(TOPK_ROWS, 128)

    top_values, top_indices = pl.pallas_call(
        _topk_kernel,
        in_specs=[pl.BlockSpec((TOPK_ROWS, 128), lambda: (0, 0))],
        out_specs=[pl.BlockSpec(memory_space=pltpu.SMEM),
                   pl.BlockSpec(memory_space=pltpu.SMEM)],
        out_shape=[jax.ShapeDtypeStruct((NUM_RETRIEVAL,), jnp.float32),
                   jax.ShapeDtypeStruct((NUM_RETRIEVAL,), jnp.int32)],
    )(padded)

    return combined_score, top_values, top_indices
